# Initial kernel scaffold; baseline (speedup 1.0000x reference)
import jax
import jax.numpy as jnp
from jax.experimental import pallas as pl


def kernel(x, W_in, b_in, W_out, b_out, embed):
    raise NotImplementedError("write your pallas kernel here")


# speedup vs baseline: 6.1230x; 6.1230x over previous
import jax
import jax.numpy as jnp
from jax import lax
from jax.experimental import pallas as pl


def _prep_kernel(embed_ref, wout_ref, bout_ref, p_ref, e2_ref):
    E = embed_ref[0]
    heads = pl.num_programs(0)
    p_ref[...] = (
        jnp.dot(E, wout_ref[...], preferred_element_type=jnp.float32)
        + (1.0 / heads) * bout_ref[...][None, :]
    )
    e2_ref[...] = jnp.sum(E * E, axis=1)[None, None, :]


def _main_kernel(x_ref, win_ref, bin_ref, embed_ref, e2_ref, p_ref,
                 out_ref, loss_ref):
    i = pl.program_id(0)
    heads, k, d = embed_ref.shape
    bn = x_ref.shape[0]
    xi = (
        jnp.dot(x_ref[...], win_ref[...], preferred_element_type=jnp.float32)
        + bin_ref[...][None, :]
    )
    out = jnp.zeros(out_ref.shape, jnp.float32)
    neg_max_sum = jnp.float32(0.0)
    iota = lax.broadcasted_iota(jnp.int32, (bn, k), 1)
    for h in range(heads):
        xi_h = xi[:, h * d:(h + 1) * d]
        dots = lax.dot_general(
            xi_h, embed_ref[h], (((1,), (1,)), ((), ())),
            preferred_element_type=jnp.float32)
        x2 = jnp.sum(xi_h * xi_h, axis=1, keepdims=True)
        dist = -((x2 - 2.0 * dots) + e2_ref[h])
        maxv = jnp.max(dist, axis=1, keepdims=True)
        idx = jnp.min(jnp.where(dist == maxv, iota, k), axis=1)
        onehot = (iota == idx[:, None]).astype(jnp.float32)
        out = out + jnp.dot(onehot, p_ref[pl.ds(h * k, k), :],
                            preferred_element_type=jnp.float32)
        neg_max_sum = neg_max_sum - jnp.sum(maxv)
    out_ref[...] = out
    prev = jnp.where(i == 0, jnp.zeros((1, 1), jnp.float32), loss_ref[...])
    loss_ref[...] = prev + neg_max_sum


def kernel(x, W_in, b_in, W_out, b_out, embed):
    n, dim = x.shape
    heads, k, d = embed.shape
    in_dim = heads * d
    bn = min(512, n)
    grid = n // bn

    P, e2 = pl.pallas_call(
        _prep_kernel,
        grid=(heads,),
        in_specs=[
            pl.BlockSpec((1, k, d), lambda h: (h, 0, 0)),
            pl.BlockSpec((d, dim), lambda h: (h, 0)),
            pl.BlockSpec((dim,), lambda h: (0,)),
        ],
        out_specs=[
            pl.BlockSpec((k, dim), lambda h: (h, 0)),
            pl.BlockSpec((1, 1, k), lambda h: (h, 0, 0)),
        ],
        out_shape=[
            jax.ShapeDtypeStruct((heads * k, dim), jnp.float32),
            jax.ShapeDtypeStruct((heads, 1, k), jnp.float32),
        ],
    )(embed, W_out, b_out)

    out, loss = pl.pallas_call(
        _main_kernel,
        grid=(grid,),
        in_specs=[
            pl.BlockSpec((bn, dim), lambda i: (i, 0)),
            pl.BlockSpec((dim, in_dim), lambda i: (0, 0)),
            pl.BlockSpec((in_dim,), lambda i: (0,)),
            pl.BlockSpec((heads, k, d), lambda i: (0, 0, 0)),
            pl.BlockSpec((heads, 1, k), lambda i: (0, 0, 0)),
            pl.BlockSpec((heads * k, dim), lambda i: (0, 0)),
        ],
        out_specs=[
            pl.BlockSpec((bn, dim), lambda i: (i, 0)),
            pl.BlockSpec((1, 1), lambda i: (0, 0)),
        ],
        out_shape=[
            jax.ShapeDtypeStruct((n, dim), jnp.float32),
            jax.ShapeDtypeStruct((1, 1), jnp.float32),
        ],
    )(x, W_in, b_in, embed, e2, P)

    l_vq = loss[0, 0] / jnp.float32(heads * n * d)
    return (out, l_vq)
